# baseline (speedup 1.0000x reference)
import jax
import jax.numpy as jnp
from jax.experimental import pallas as pl
from jax.experimental.pallas import tpu as pltpu

M, B, D = 65536, 8192, 256
CH = 8192
NCH = M // CH
NSEM = 2 * NCH + 2


def _dma_body(k, v, nk, nv, ok, ov, sems):
    copies = []
    idx = 0
    for c in range(NCH):
        copies.append(pltpu.make_async_copy(
            k.at[pl.ds(c * CH, CH), :], ok.at[pl.ds(c * CH, CH), :],
            sems.at[idx]))
        idx += 1
        copies.append(pltpu.make_async_copy(
            v.at[pl.ds(c * CH, CH), :], ov.at[pl.ds(c * CH, CH), :],
            sems.at[idx]))
        idx += 1
    copies.append(pltpu.make_async_copy(
        nk, ok.at[pl.ds(M, B), :], sems.at[idx]))
    idx += 1
    copies.append(pltpu.make_async_copy(
        nv, ov.at[pl.ds(M, B), :], sems.at[idx]))
    for c in copies:
        c.start()
    for c in copies:
        c.wait()


def kernel(keys, vals, new_keys, new_vals):
    hbm = pl.BlockSpec(memory_space=pltpu.MemorySpace.HBM)
    out_shape = jax.ShapeDtypeStruct((M + B, D), keys.dtype)
    return pl.pallas_call(
        _dma_body,
        in_specs=[hbm, hbm, hbm, hbm],
        out_specs=[hbm, hbm],
        out_shape=[out_shape, out_shape],
        scratch_shapes=[pltpu.SemaphoreType.DMA((NSEM,))],
    )(keys, vals, new_keys, new_vals)


# speedup vs baseline: 48.6753x; 48.6753x over previous
import jax
import jax.numpy as jnp
from jax.experimental import pallas as pl

M, B, D = 65536, 8192, 256
BLK = 4096
NM = M // BLK
NB = B // BLK


def _copy_body(k_ref, v_ref, nk_ref, nv_ref, ok_ref, ov_ref):
    i = pl.program_id(0)

    @pl.when(i < NM)
    def _():
        ok_ref[...] = k_ref[...]
        ov_ref[...] = v_ref[...]

    @pl.when(i >= NM)
    def _():
        ok_ref[...] = nk_ref[...]
        ov_ref[...] = nv_ref[...]


def kernel(keys, vals, new_keys, new_vals):
    grid = (NM + NB,)
    old_spec = pl.BlockSpec((BLK, D), lambda i: (jnp.minimum(i, NM - 1), 0))
    new_spec = pl.BlockSpec((BLK, D), lambda i: (jnp.maximum(i - NM, 0), 0))
    out_spec = pl.BlockSpec((BLK, D), lambda i: (i, 0))
    out_shape = jax.ShapeDtypeStruct((M + B, D), keys.dtype)
    return pl.pallas_call(
        _copy_body,
        grid=grid,
        in_specs=[old_spec, old_spec, new_spec, new_spec],
        out_specs=[out_spec, out_spec],
        out_shape=[out_shape, out_shape],
    )(keys, vals, new_keys, new_vals)
